# Initial kernel scaffold; baseline (speedup 1.0000x reference)
import jax
import jax.numpy as jnp
from jax.experimental import pallas as pl


def kernel(inputs):
    raise NotImplementedError("write your pallas kernel here")


# speedup vs baseline: 5.6238x; 5.6238x over previous
import functools

import jax
import jax.numpy as jnp
from jax import lax
from jax.experimental import pallas as pl
from jax.experimental.pallas import tpu as pltpu
from jax.experimental.pallas import tpu_sc as plsc

_ROWS = 64
_N = 8192
_K1 = 129
_L = 16
_NV = _N // _L
_NC = 2
_NS = 16
_NW = _NC * _NS
_RPW = _ROWS // _NW

_mesh = plsc.VectorSubcoreMesh(core_axis_name="c", subcore_axis_name="s")


@functools.partial(
    pl.kernel,
    out_type=jax.ShapeDtypeStruct((_ROWS, _N), jnp.float32),
    mesh=_mesh,
    compiler_params=pltpu.CompilerParams(needs_layout_passes=False),
    scratch_types=[
        pltpu.VMEM((_RPW, _N), jnp.float32),
        pltpu.VMEM((_RPW, _N), jnp.int32),
        pltpu.VMEM((_RPW, _N), jnp.float32),
    ],
)
def _ksparse_kernel(x_hbm, out_hbm, rows_v, keys_v, outs_v):
    wid = lax.axis_index("s") * _NC + lax.axis_index("c")
    base = wid * _RPW
    pltpu.sync_copy(x_hbm.at[pl.ds(base, _RPW)], rows_v)

    zero_i = jnp.zeros((_L,), jnp.int32)
    one_i = jnp.ones((_L,), jnp.int32)
    k1_v = jnp.full((_L,), _K1, jnp.int32)
    min_v = jnp.full((_L,), -(2 ** 31), jnp.int32)
    zero_f = jnp.zeros((_L,), jnp.float32)

    for r in range(_RPW):
        def key_body(j, _):
            x = rows_v[r, pl.ds(j * _L, _L)]
            b = lax.bitcast_convert_type(x, jnp.int32)
            neg = lax.shift_right_arithmetic(b, 31)
            keys_v[r, pl.ds(j * _L, _L)] = b ^ (neg & jnp.int32(0x7FFFFFFF))
            return 0

        lax.fori_loop(0, _NV, key_body, 0, unroll=8)

        def count_ge(tvec):
            def body(j, acc):
                k = keys_v[r, pl.ds(j * _L, _L)]
                return acc + plsc.all_reduce_population_count(k >= tvec)

            return lax.fori_loop(0, _NV, body, zero_i, unroll=8)

        tv = jnp.where(count_ge(zero_i) >= k1_v, zero_i, min_v)

        def bit_body(i, tv):
            bit_v = lax.broadcast(jnp.int32(30) - i, (_L,))
            tent = tv + lax.shift_left(one_i, bit_v)
            return jnp.where(count_ge(tent) >= k1_v, tent, tv)

        tv = lax.fori_loop(0, 31, bit_body, tv)

        def mask_body(j, _):
            k = keys_v[r, pl.ds(j * _L, _L)]
            x = rows_v[r, pl.ds(j * _L, _L)]
            outs_v[r, pl.ds(j * _L, _L)] = jnp.where(k > tv, x, zero_f)
            return 0

        lax.fori_loop(0, _NV, mask_body, 0, unroll=8)

    pltpu.sync_copy(outs_v, out_hbm.at[pl.ds(base, _RPW)])


def kernel(inputs):
    return _ksparse_kernel(inputs)
